# baseline (speedup 1.0000x reference)
import functools

import jax
import jax.numpy as jnp
from jax import lax
from jax.experimental import pallas as pl
from jax.experimental.pallas import tpu as pltpu
from jax.experimental.pallas import tpu_sc as plsc

VOCAB = 100000
DIM = 64
B = 16384
K = 20

NC = 2
NS = 16
NW = NC * NS
BPW = B // NW
CB = 32
NCHUNK = BPW // CB


def _sc_body(tw_hbm, cw_hbm, nw_hbm, tt_hbm, ct_hbm,
             pos_hbm, neg_hbm,
             ti_v, ci_v, ni_v, po_v, no_v, tp_v,
             tr0, cr0, nr0, tr1, cr1, nr1, sem0, sem1):
    wid = lax.axis_index("s") * NC + lax.axis_index("c")
    base = wid * BPW
    pltpu.sync_copy(tw_hbm.at[pl.ds(base, BPW)], ti_v)
    pltpu.sync_copy(cw_hbm.at[pl.ds(base, BPW)], ci_v)
    pltpu.sync_copy(nw_hbm.at[pl.ds(base * K, BPW * K)], ni_v)

    bufs = ((tr0, cr0, nr0, sem0), (tr1, cr1, nr1, sem1))
    lanes = lax.iota(jnp.int32, 16)

    def issue(c, slot):
        tr, cr, nr, sem = bufs[slot]
        o = c * CB
        pltpu.async_copy(tt_hbm.at[ti_v.at[pl.ds(o, CB)]], tr, sem)
        pltpu.async_copy(ct_hbm.at[ci_v.at[pl.ds(o, CB)]], cr, sem)
        pltpu.async_copy(ct_hbm.at[ni_v.at[pl.ds(o * K, CB * K)]], nr, sem)

    def drain(slot):
        tr, cr, nr, sem = bufs[slot]
        pltpu.make_async_copy(tt_hbm.at[pl.ds(0, CB)], tr, sem).wait()
        pltpu.make_async_copy(ct_hbm.at[pl.ds(0, CB)], cr, sem).wait()
        pltpu.make_async_copy(ct_hbm.at[pl.ds(0, CB * K)], nr, sem).wait()

    lanes16 = lanes * 16

    def compute(c, slot):
        tr, cr, nr, _ = bufs[slot]

        def quad(ref, r):
            return [ref[r, pl.ds(16 * j, 16)] for j in range(4)]

        def dot4(q, t):
            return (q[0] * t[0] + q[1] * t[1]) + (q[2] * t[2] + q[3] * t[3])

        def loads(s, b):
            return quad(nr, b * K + s) if s < K else quad(cr, b)

        NCH = 4

        def body(i, carry):
            b0 = i * NCH
            g0 = c * CB + b0
            ts = [quad(tr, b0 + u) for u in range(NCH)]
            cur = [loads(0, b0 + u) for u in range(NCH)]
            for s in range(K + 1):
                if s + 1 <= K:
                    nxt = [loads(s + 1, b0 + u) for u in range(NCH)]
                for u in range(NCH):
                    tp_v[pl.ds(u * 512 + s * 16, 16)] = dot4(cur[u], ts[u])
                cur = nxt
            bases = tuple(u * 512 + h for u in range(NCH) for h in (0, 256))
            accs = [plsc.load_gather(tp_v, [lanes16 + bb]) for bb in bases]
            for j in range(1, 16):
                accs = [acc + plsc.load_gather(tp_v, [lanes16 + (bb + j)])
                        for acc, bb in zip(accs, bases)]
            accs = [jnp.clip(a, -10.0, 10.0) for a in accs]
            pidx = jnp.full((16,), g0, jnp.int32)
            for u in range(NCH):
                sa, sb = accs[2 * u], accs[2 * u + 1]
                plsc.store_scatter(no_v, [pidx + u, lanes], sa)
                plsc.store_scatter(no_v, [pidx + u, lanes + 16], sb,
                                   mask=lanes < (K - 16))
                plsc.store_scatter(po_v, [pidx + u], sb,
                                   mask=lanes == (K - 16))
            return carry

        lax.fori_loop(0, CB // NCH, body, 0)

    issue(0, 0)

    def pair(i, carry):
        g = i * 2
        issue(g + 1, 1)
        drain(0)
        compute(g, 0)

        @pl.when(g + 2 < NCHUNK)
        def _():
            issue(g + 2, 0)

        drain(1)
        compute(g + 1, 1)
        return carry

    lax.fori_loop(0, NCHUNK // 2, pair, 0)

    pltpu.sync_copy(po_v, pos_hbm.at[pl.ds(base, BPW)])
    pltpu.sync_copy(no_v.at[pl.ds(0, BPW)], neg_hbm.at[pl.ds(base, BPW)])


_sc_call = functools.partial(
    pl.kernel,
    out_type=[
        jax.ShapeDtypeStruct((B,), jnp.float32),
        jax.ShapeDtypeStruct((B, K), jnp.float32),
    ],
    mesh=plsc.VectorSubcoreMesh(core_axis_name="c", subcore_axis_name="s"),
    compiler_params=pltpu.CompilerParams(needs_layout_passes=False,
                                         use_tc_tiling_on_sc=False),
    scratch_types=[
        pltpu.VMEM((BPW,), jnp.int32),
        pltpu.VMEM((BPW,), jnp.int32),
        pltpu.VMEM((BPW * K,), jnp.int32),
        pltpu.VMEM((BPW,), jnp.float32),
        pltpu.VMEM((BPW + 1, K), jnp.float32),
        pltpu.VMEM((2048,), jnp.float32),
        pltpu.VMEM((CB, DIM), jnp.float32),
        pltpu.VMEM((CB, DIM), jnp.float32),
        pltpu.VMEM((CB * K, DIM), jnp.float32),
        pltpu.VMEM((CB, DIM), jnp.float32),
        pltpu.VMEM((CB, DIM), jnp.float32),
        pltpu.VMEM((CB * K, DIM), jnp.float32),
        pltpu.SemaphoreType.DMA,
        pltpu.SemaphoreType.DMA,
    ],
)(_sc_body)


def kernel(target_word, context_word, negative_words, target_table, context_table):
    neg_flat = negative_words.reshape(-1).astype(jnp.int32)
    pos, neg = _sc_call(
        target_word.astype(jnp.int32),
        context_word.astype(jnp.int32),
        neg_flat,
        target_table,
        context_table,
    )
    return pos, neg


# speedup vs baseline: 1.2542x; 1.2542x over previous
import functools

import jax
import jax.numpy as jnp
from jax import lax
from jax.experimental import pallas as pl
from jax.experimental.pallas import tpu as pltpu
from jax.experimental.pallas import tpu_sc as plsc

VOCAB = 100000
DIM = 64
B = 16384
K = 20

NC = 2
NS = 16
NW = NC * NS
BPW = B // NW
CB = 32
NCHUNK = BPW // CB


def _sc_body(tw_hbm, cw_hbm, nw_hbm, tt_hbm, ct_hbm,
             pos_hbm, neg_hbm,
             ti_v, ci_v, ni_v, po_v, no_v, tp_v,
             tr0, cr0, nr0, tr1, cr1, nr1, sem0, sem1):
    wid = lax.axis_index("s") * NC + lax.axis_index("c")
    base = wid * BPW
    pltpu.sync_copy(tw_hbm.at[pl.ds(base, BPW)], ti_v)
    pltpu.sync_copy(cw_hbm.at[pl.ds(base, BPW)], ci_v)
    pltpu.sync_copy(nw_hbm.at[pl.ds(base * K, BPW * K)], ni_v)

    bufs = ((tr0, cr0, nr0, sem0), (tr1, cr1, nr1, sem1))
    lanes = lax.iota(jnp.int32, 16)

    def issue(c, slot):
        tr, cr, nr, sem = bufs[slot]
        o = c * CB
        pltpu.async_copy(tt_hbm.at[ti_v.at[pl.ds(o, CB)]], tr, sem)
        pltpu.async_copy(ct_hbm.at[ci_v.at[pl.ds(o, CB)]], cr, sem)
        pltpu.async_copy(ct_hbm.at[ni_v.at[pl.ds(o * K, CB * K)]], nr, sem)

    def drain(slot):
        tr, cr, nr, sem = bufs[slot]
        pltpu.make_async_copy(tt_hbm.at[pl.ds(0, CB)], tr, sem).wait()
        pltpu.make_async_copy(ct_hbm.at[pl.ds(0, CB)], cr, sem).wait()
        pltpu.make_async_copy(ct_hbm.at[pl.ds(0, CB * K)], nr, sem).wait()

    lanes16 = lanes * 16

    def compute(c, slot):
        tr, cr, nr, _ = bufs[slot]

        def quad(ref, r):
            return [ref[r, pl.ds(16 * j, 16)] for j in range(4)]

        def dot4(q, t):
            return (q[0] * t[0] + q[1] * t[1]) + (q[2] * t[2] + q[3] * t[3])

        def loads(s, b):
            return quad(nr, b * K + s) if s < K else quad(cr, b)

        NCH = 2
        perms = [jnp.bitwise_xor(lanes, 1 << e)[:, None] for e in range(4)]
        dnums = lax.GatherDimensionNumbers(
            offset_dims=(), collapsed_slice_dims=(0,), start_index_map=(0,))

        def hsum(p):
            for perm in perms:
                p = p + lax.gather(p, perm, dnums, (1,),
                                   mode=lax.GatherScatterMode.PROMISE_IN_BOUNDS)
            return p

        def body(i, carry):
            b0 = i * NCH
            g0 = c * CB + b0
            ts = [quad(tr, b0 + u) for u in range(NCH)]
            cur = [loads(0, b0 + u) for u in range(NCH)]
            sa = [jnp.zeros((16,), jnp.float32) for _ in range(NCH)]
            sb = [jnp.zeros((16,), jnp.float32) for _ in range(NCH)]
            for s in range(K + 1):
                if s + 1 <= K:
                    nxt = [loads(s + 1, b0 + u) for u in range(NCH)]
                for u in range(NCH):
                    r = hsum(dot4(cur[u], ts[u]))
                    if s < 16:
                        sa[u] = jnp.where(lanes == s, r, sa[u])
                    else:
                        sb[u] = jnp.where(lanes == (s - 16), r, sb[u])
                cur = nxt
            pidx = jnp.full((16,), g0, jnp.int32)
            for u in range(NCH):
                plsc.store_scatter(no_v, [pidx + u, lanes],
                                   jnp.clip(sa[u], -10.0, 10.0))
                sbu = jnp.clip(sb[u], -10.0, 10.0)
                plsc.store_scatter(no_v, [pidx + u, lanes + 16], sbu,
                                   mask=lanes < (K - 16))
                plsc.store_scatter(po_v, [pidx + u], sbu,
                                   mask=lanes == (K - 16))
            return carry

        lax.fori_loop(0, CB // NCH, body, 0)

    issue(0, 0)

    def pair(i, carry):
        g = i * 2
        issue(g + 1, 1)
        drain(0)
        compute(g, 0)

        @pl.when(g + 2 < NCHUNK)
        def _():
            issue(g + 2, 0)

        drain(1)
        compute(g + 1, 1)
        return carry

    lax.fori_loop(0, NCHUNK // 2, pair, 0)

    pltpu.sync_copy(po_v, pos_hbm.at[pl.ds(base, BPW)])
    pltpu.sync_copy(no_v.at[pl.ds(0, BPW)], neg_hbm.at[pl.ds(base, BPW)])


_sc_call = functools.partial(
    pl.kernel,
    out_type=[
        jax.ShapeDtypeStruct((B,), jnp.float32),
        jax.ShapeDtypeStruct((B, K), jnp.float32),
    ],
    mesh=plsc.VectorSubcoreMesh(core_axis_name="c", subcore_axis_name="s"),
    compiler_params=pltpu.CompilerParams(needs_layout_passes=False,
                                         use_tc_tiling_on_sc=False),
    scratch_types=[
        pltpu.VMEM((BPW,), jnp.int32),
        pltpu.VMEM((BPW,), jnp.int32),
        pltpu.VMEM((BPW * K,), jnp.int32),
        pltpu.VMEM((BPW,), jnp.float32),
        pltpu.VMEM((BPW + 1, K), jnp.float32),
        pltpu.VMEM((2048,), jnp.float32),
        pltpu.VMEM((CB, DIM), jnp.float32),
        pltpu.VMEM((CB, DIM), jnp.float32),
        pltpu.VMEM((CB * K, DIM), jnp.float32),
        pltpu.VMEM((CB, DIM), jnp.float32),
        pltpu.VMEM((CB, DIM), jnp.float32),
        pltpu.VMEM((CB * K, DIM), jnp.float32),
        pltpu.SemaphoreType.DMA,
        pltpu.SemaphoreType.DMA,
    ],
)(_sc_body)


def kernel(target_word, context_word, negative_words, target_table, context_table):
    neg_flat = negative_words.reshape(-1).astype(jnp.int32)
    pos, neg = _sc_call(
        target_word.astype(jnp.int32),
        context_word.astype(jnp.int32),
        neg_flat,
        target_table,
        context_table,
    )
    return pos, neg


# speedup vs baseline: 1.2605x; 1.0050x over previous
import functools

import jax
import jax.numpy as jnp
from jax import lax
from jax.experimental import pallas as pl
from jax.experimental.pallas import tpu as pltpu
from jax.experimental.pallas import tpu_sc as plsc

VOCAB = 100000
DIM = 64
B = 16384
K = 20

NC = 2
NS = 16
NW = NC * NS
BPW = B // NW
CB = 32
NCHUNK = BPW // CB


def _sc_body(tw_hbm, cw_hbm, nw_hbm, tt_hbm, ct_hbm,
             pos_hbm, neg_hbm,
             ti_v, ci_v, ni_v, po_v, no_v,
             tr0, cr0, nr0, tr1, cr1, nr1, sem0, sem1):
    wid = lax.axis_index("s") * NC + lax.axis_index("c")
    base = wid * BPW
    cp1 = pltpu.async_copy(tw_hbm.at[pl.ds(base, BPW)], ti_v, sem0)
    cp2 = pltpu.async_copy(cw_hbm.at[pl.ds(base, BPW)], ci_v, sem0)
    cp3 = pltpu.async_copy(nw_hbm.at[pl.ds(base * K, BPW * K)], ni_v, sem0)
    cp1.wait()
    cp2.wait()
    cp3.wait()

    bufs = ((tr0, cr0, nr0, sem0), (tr1, cr1, nr1, sem1))
    lanes = lax.iota(jnp.int32, 16)

    def issue(c, slot):
        tr, cr, nr, sem = bufs[slot]
        o = c * CB
        pltpu.async_copy(tt_hbm.at[ti_v.at[pl.ds(o, CB)]], tr, sem)
        pltpu.async_copy(ct_hbm.at[ci_v.at[pl.ds(o, CB)]], cr, sem)
        pltpu.async_copy(ct_hbm.at[ni_v.at[pl.ds(o * K, CB * K)]], nr, sem)

    def drain(slot):
        tr, cr, nr, sem = bufs[slot]
        pltpu.make_async_copy(tt_hbm.at[pl.ds(0, CB)], tr, sem).wait()
        pltpu.make_async_copy(ct_hbm.at[pl.ds(0, CB)], cr, sem).wait()
        pltpu.make_async_copy(ct_hbm.at[pl.ds(0, CB * K)], nr, sem).wait()

    def compute(c, slot):
        tr, cr, nr, _ = bufs[slot]

        def quad(ref, r):
            return [ref[r, pl.ds(16 * j, 16)] for j in range(4)]

        def dot4(q, t):
            return (q[0] * t[0] + q[1] * t[1]) + (q[2] * t[2] + q[3] * t[3])

        def loads(s, b):
            return quad(nr, b * K + s) if s < K else quad(cr, b)

        NCH = 2
        perms = [jnp.bitwise_xor(lanes, 1 << e)[:, None] for e in range(4)]
        dnums = lax.GatherDimensionNumbers(
            offset_dims=(), collapsed_slice_dims=(0,), start_index_map=(0,))

        def hsum(p):
            for perm in perms:
                p = p + lax.gather(p, perm, dnums, (1,),
                                   mode=lax.GatherScatterMode.PROMISE_IN_BOUNDS)
            return p

        def body(i, carry):
            b0 = i * NCH
            g0 = c * CB + b0
            ts = [quad(tr, b0 + u) for u in range(NCH)]
            cur = [loads(0, b0 + u) for u in range(NCH)]
            sa = [jnp.zeros((16,), jnp.float32) for _ in range(NCH)]
            sb = [jnp.zeros((16,), jnp.float32) for _ in range(NCH)]
            for s in range(K + 1):
                if s + 1 <= K:
                    nxt = [loads(s + 1, b0 + u) for u in range(NCH)]
                for u in range(NCH):
                    r = hsum(dot4(cur[u], ts[u]))
                    if s < 16:
                        sa[u] = jnp.where(lanes == s, r, sa[u])
                    else:
                        sb[u] = jnp.where(lanes == (s - 16), r, sb[u])
                cur = nxt
            pidx = jnp.full((16,), g0, jnp.int32)
            for u in range(NCH):
                plsc.store_scatter(no_v, [pidx + u, lanes],
                                   jnp.clip(sa[u], -10.0, 10.0))
                sbu = jnp.clip(sb[u], -10.0, 10.0)
                plsc.store_scatter(no_v, [pidx + u, lanes + 16], sbu,
                                   mask=lanes < (K - 16))
                plsc.store_scatter(po_v, [pidx + u], sbu,
                                   mask=lanes == (K - 16))
            return carry

        lax.fori_loop(0, CB // NCH, body, 0)

    issue(0, 0)

    def pair(i, carry):
        g = i * 2
        issue(g + 1, 1)
        drain(0)
        compute(g, 0)

        @pl.when(g + 2 < NCHUNK)
        def _():
            issue(g + 2, 0)

        drain(1)
        compute(g + 1, 1)
        return carry

    lax.fori_loop(0, NCHUNK // 2, pair, 0)

    pltpu.sync_copy(po_v, pos_hbm.at[pl.ds(base, BPW)])
    pltpu.sync_copy(no_v.at[pl.ds(0, BPW)], neg_hbm.at[pl.ds(base, BPW)])


_sc_call = functools.partial(
    pl.kernel,
    out_type=[
        jax.ShapeDtypeStruct((B,), jnp.float32),
        jax.ShapeDtypeStruct((B, K), jnp.float32),
    ],
    mesh=plsc.VectorSubcoreMesh(core_axis_name="c", subcore_axis_name="s"),
    compiler_params=pltpu.CompilerParams(needs_layout_passes=False,
                                         use_tc_tiling_on_sc=False),
    scratch_types=[
        pltpu.VMEM((BPW,), jnp.int32),
        pltpu.VMEM((BPW,), jnp.int32),
        pltpu.VMEM((BPW * K,), jnp.int32),
        pltpu.VMEM((BPW,), jnp.float32),
        pltpu.VMEM((BPW + 1, K), jnp.float32),
        pltpu.VMEM((CB, DIM), jnp.float32),
        pltpu.VMEM((CB, DIM), jnp.float32),
        pltpu.VMEM((CB * K, DIM), jnp.float32),
        pltpu.VMEM((CB, DIM), jnp.float32),
        pltpu.VMEM((CB, DIM), jnp.float32),
        pltpu.VMEM((CB * K, DIM), jnp.float32),
        pltpu.SemaphoreType.DMA,
        pltpu.SemaphoreType.DMA,
    ],
)(_sc_body)


def kernel(target_word, context_word, negative_words, target_table, context_table):
    neg_flat = negative_words.reshape(-1).astype(jnp.int32)
    pos, neg = _sc_call(
        target_word.astype(jnp.int32),
        context_word.astype(jnp.int32),
        neg_flat,
        target_table,
        context_table,
    )
    return pos, neg


# speedup vs baseline: 1.2639x; 1.0027x over previous
import functools

import jax
import jax.numpy as jnp
from jax import lax
from jax.experimental import pallas as pl
from jax.experimental.pallas import tpu as pltpu
from jax.experimental.pallas import tpu_sc as plsc

VOCAB = 100000
DIM = 64
B = 16384
K = 20

NC = 2
NS = 16
NW = NC * NS
BPW = B // NW
CB = 32
NCHUNK = BPW // CB


def _sc_body(tw_hbm, cw_hbm, nw_hbm, tt_hbm, ct_hbm,
             pos_hbm, neg_hbm,
             ti_v, ci_v, ni_v, po_v, no_v,
             tr0, cr0, nr0, tr1, cr1, nr1, sem0, sem1):
    wid = lax.axis_index("s") * NC + lax.axis_index("c")
    base = wid * BPW
    cp1 = pltpu.async_copy(tw_hbm.at[pl.ds(base, BPW)], ti_v, sem0)
    cp2 = pltpu.async_copy(cw_hbm.at[pl.ds(base, BPW)], ci_v, sem0)
    cp3 = pltpu.async_copy(nw_hbm.at[pl.ds(base * K, BPW * K)], ni_v, sem0)
    cp1.wait()
    cp2.wait()
    cp3.wait()

    bufs = ((tr0, cr0, nr0, sem0), (tr1, cr1, nr1, sem1))
    lanes = lax.iota(jnp.int32, 16)

    def issue(c, slot):
        tr, cr, nr, sem = bufs[slot]
        o = c * CB
        pltpu.async_copy(tt_hbm.at[ti_v.at[pl.ds(o, CB)]], tr, sem)
        pltpu.async_copy(ct_hbm.at[ci_v.at[pl.ds(o, CB)]], cr, sem)
        pltpu.async_copy(ct_hbm.at[ni_v.at[pl.ds(o * K, CB * K)]], nr, sem)

    def drain(slot):
        tr, cr, nr, sem = bufs[slot]
        pltpu.make_async_copy(tt_hbm.at[pl.ds(0, CB)], tr, sem).wait()
        pltpu.make_async_copy(ct_hbm.at[pl.ds(0, CB)], cr, sem).wait()
        pltpu.make_async_copy(ct_hbm.at[pl.ds(0, CB * K)], nr, sem).wait()

    def compute(c, slot):
        tr, cr, nr, _ = bufs[slot]

        def quad(ref, r):
            return [ref[r, pl.ds(16 * j, 16)] for j in range(4)]

        def dot4(q, t):
            return (q[0] * t[0] + q[1] * t[1]) + (q[2] * t[2] + q[3] * t[3])

        def loads(s, b):
            return quad(nr, b * K + s) if s < K else quad(cr, b)

        NCH = 4
        perms = [jnp.bitwise_xor(lanes, 1 << e)[:, None] for e in range(4)]
        dnums = lax.GatherDimensionNumbers(
            offset_dims=(), collapsed_slice_dims=(0,), start_index_map=(0,))

        def hsum(p):
            for perm in perms:
                p = p + lax.gather(p, perm, dnums, (1,),
                                   mode=lax.GatherScatterMode.PROMISE_IN_BOUNDS)
            return p

        def body(i, carry):
            b0 = i * NCH
            g0 = c * CB + b0
            ts = [quad(tr, b0 + u) for u in range(NCH)]
            cur = [loads(0, b0 + u) for u in range(NCH)]
            sa = [jnp.zeros((16,), jnp.float32) for _ in range(NCH)]
            sb = [jnp.zeros((16,), jnp.float32) for _ in range(NCH)]
            for s in range(K + 1):
                if s + 1 <= K:
                    nxt = [loads(s + 1, b0 + u) for u in range(NCH)]
                for u in range(NCH):
                    r = hsum(dot4(cur[u], ts[u]))
                    if s < 16:
                        sa[u] = jnp.where(lanes == s, r, sa[u])
                    else:
                        sb[u] = jnp.where(lanes == (s - 16), r, sb[u])
                cur = nxt
            pidx = jnp.full((16,), g0, jnp.int32)
            for u in range(NCH):
                plsc.store_scatter(no_v, [pidx + u, lanes],
                                   jnp.clip(sa[u], -10.0, 10.0))
                sbu = jnp.clip(sb[u], -10.0, 10.0)
                plsc.store_scatter(no_v, [pidx + u, lanes + 16], sbu,
                                   mask=lanes < (K - 16))
                plsc.store_scatter(po_v, [pidx + u], sbu,
                                   mask=lanes == (K - 16))
            return carry

        lax.fori_loop(0, CB // NCH, body, 0)

    issue(0, 0)

    def pair(i, carry):
        g = i * 2
        issue(g + 1, 1)
        drain(0)
        compute(g, 0)

        @pl.when(g + 2 < NCHUNK)
        def _():
            issue(g + 2, 0)

        drain(1)
        compute(g + 1, 1)
        return carry

    lax.fori_loop(0, NCHUNK // 2, pair, 0)

    pltpu.sync_copy(po_v, pos_hbm.at[pl.ds(base, BPW)])
    pltpu.sync_copy(no_v.at[pl.ds(0, BPW)], neg_hbm.at[pl.ds(base, BPW)])


_sc_call = functools.partial(
    pl.kernel,
    out_type=[
        jax.ShapeDtypeStruct((B,), jnp.float32),
        jax.ShapeDtypeStruct((B, K), jnp.float32),
    ],
    mesh=plsc.VectorSubcoreMesh(core_axis_name="c", subcore_axis_name="s"),
    compiler_params=pltpu.CompilerParams(needs_layout_passes=False,
                                         use_tc_tiling_on_sc=False),
    scratch_types=[
        pltpu.VMEM((BPW,), jnp.int32),
        pltpu.VMEM((BPW,), jnp.int32),
        pltpu.VMEM((BPW * K,), jnp.int32),
        pltpu.VMEM((BPW,), jnp.float32),
        pltpu.VMEM((BPW + 1, K), jnp.float32),
        pltpu.VMEM((CB, DIM), jnp.float32),
        pltpu.VMEM((CB, DIM), jnp.float32),
        pltpu.VMEM((CB * K, DIM), jnp.float32),
        pltpu.VMEM((CB, DIM), jnp.float32),
        pltpu.VMEM((CB, DIM), jnp.float32),
        pltpu.VMEM((CB * K, DIM), jnp.float32),
        pltpu.SemaphoreType.DMA,
        pltpu.SemaphoreType.DMA,
    ],
)(_sc_body)


def kernel(target_word, context_word, negative_words, target_table, context_table):
    neg_flat = negative_words.reshape(-1).astype(jnp.int32)
    pos, neg = _sc_call(
        target_word.astype(jnp.int32),
        context_word.astype(jnp.int32),
        neg_flat,
        target_table,
        context_table,
    )
    return pos, neg
